# Initial kernel scaffold; baseline (speedup 1.0000x reference)
import jax
import jax.numpy as jnp
from jax.experimental import pallas as pl


def kernel(ed, fa, ad, W_e1, W_e2, W_f1, W_f2, W_mc0, W_mc1, W_mc2, W_pb, W_fn0, b_fn0, W_fn1, b_fn1, W_fn2, b_fn2):
    raise NotImplementedError("write your pallas kernel here")


# speedup vs baseline: 8.8920x; 8.8920x over previous
import functools

import jax
import jax.numpy as jnp
from jax import lax
from jax.experimental import pallas as pl
from jax.experimental.pallas import tpu as pltpu
from jax.experimental.pallas import tpu_sc as plsc

F32 = jnp.float32
EPS = 1e-5
FT = 2000


def _dot(a, b):
    return lax.dot_general(a, b, (((1,), (0,)), ((), ())),
                           preferred_element_type=F32)


def _sumsq_row(y):
    p = jnp.concatenate([y, y * y], axis=1)
    return jnp.sum(p, axis=0, keepdims=True)


def _head_s1_body(e_ref, f_ref, we1_ref, wf1_ref, o_ref):
    y = jnp.concatenate([_dot(e_ref[...], we1_ref[...]),
                         _dot(f_ref[...], wf1_ref[...])], axis=1)
    o_ref[0] = _sumsq_row(y)


def _head_s2_body(e_ref, f_ref, a1_ref, we1_ref, wf1_ref, we2_ref, wf2_ref, o_ref):
    x1e = jnp.maximum(_dot(e_ref[...], we1_ref[...]) * a1_ref[0:1] + a1_ref[1:2], 0.0)
    x1f = jnp.maximum(_dot(f_ref[...], wf1_ref[...]) * a1_ref[2:3] + a1_ref[3:4], 0.0)
    y = jnp.concatenate([_dot(x1e, we2_ref[...]), _dot(x1f, wf2_ref[...])], axis=1)
    o_ref[0] = _sumsq_row(y)


def _head_z_body(e_ref, f_ref, a1_ref, a2_ref, we1_ref, wf1_ref,
                 we2_ref, wf2_ref, wt_ref, z0_ref, z1_ref, z2_ref, z3_ref):
    x1e = jnp.maximum(_dot(e_ref[...], we1_ref[...]) * a1_ref[0:1] + a1_ref[1:2], 0.0)
    x1f = jnp.maximum(_dot(f_ref[...], wf1_ref[...]) * a1_ref[2:3] + a1_ref[3:4], 0.0)
    x2e = jnp.maximum(_dot(x1e, we2_ref[...]) * a2_ref[0:1] + a2_ref[1:2], 0.0)
    x2f = jnp.maximum(_dot(x1f, wf2_ref[...]) * a2_ref[2:3] + a2_ref[3:4], 0.0)
    a = jnp.concatenate([x2e, x2f], axis=1)
    z0_ref[...] = _dot(a, wt_ref[0])
    z1_ref[...] = _dot(a, wt_ref[1])
    z2_ref[...] = _dot(a, wt_ref[2])
    z3_ref[...] = _dot(a, wt_ref[3])


def _tc_z_body(y_ref, aff_ref, wt_ref, z0_ref, z1_ref, z2_ref, z3_ref):
    a = jnp.maximum(y_ref[...] * aff_ref[0:1] + aff_ref[1:2], 0.0)
    z0_ref[...] = _dot(a, wt_ref[0])
    z1_ref[...] = _dot(a, wt_ref[1])
    z2_ref[...] = _dot(a, wt_ref[2])
    z3_ref[...] = _dot(a, wt_ref[3])


def _pool_body(y_ref, aff_ref, wpb_ref, o_ref):
    a = jnp.maximum(y_ref[...] * aff_ref[0:1] + aff_ref[1:2], 0.0)
    ypb = _dot(a, wpb_ref[...])
    o_ref[0] = _sumsq_row(ypb)


def _fc_body(nf_ref, sb_ref, q_ref, w0_ref, b0_ref, w1_ref, b1_ref,
             w2_ref, b2_ref, o_ref):
    inv_f = nf_ref[0, 0]
    inv_n = nf_ref[0, 1]
    sb = sb_ref[...]
    t = sb * inv_f
    m = jnp.sum(sb, axis=0, keepdims=True) * inv_n
    var = q_ref[...] * inv_n - m * m
    pooled = (t - m) * lax.rsqrt(var + EPS)
    h = jnp.maximum(lax.dot_general(pooled, w0_ref[...], (((1,), (1,)), ((), ())),
                                    preferred_element_type=F32) + b0_ref[...], 0.0)
    h = jnp.maximum(lax.dot_general(h, w1_ref[...], (((1,), (1,)), ((), ())),
                                    preferred_element_type=F32) + b1_ref[...], 0.0)
    o_ref[...] = lax.dot_general(h, w2_ref[...], (((1,), (1,)), ((), ())),
                                 preferred_element_type=F32) + b2_ref[...]


def _make_sc_combine(N):
    NW = 32
    RC = 128
    nchunk = N // RC
    iters = (nchunk + NW - 1) // NW
    mesh = plsc.VectorSubcoreMesh(core_axis_name="c", subcore_axis_name="s")

    @functools.partial(
        pl.kernel, mesh=mesh,
        compiler_params=pltpu.CompilerParams(use_tc_tiling_on_sc=False),
        out_type=(jax.ShapeDtypeStruct((N, 64), F32),
                  jax.ShapeDtypeStruct((NW, 128), F32)),
        scratch_types=[pltpu.VMEM((RC,), jnp.int32),
                       pltpu.VMEM((RC,), jnp.int32),
                       pltpu.VMEM((RC,), jnp.int32),
                       pltpu.VMEM((RC, 64), F32),
                       pltpu.VMEM((RC, 64), F32),
                       pltpu.VMEM((RC, 64), F32),
                       pltpu.VMEM((RC, 64), F32),
                       pltpu.VMEM((RC, 64), F32),
                       pltpu.VMEM((128,), F32),
                       pltpu.SemaphoreType.DMA,
                       pltpu.SemaphoreType.DMA],
    )
    def sck(z0h, z1h, z2h, z3h, i0h, i1h, i2h, yh, sth,
            ib0, ib1, ib2, b0, b1, b2, b3, yb, accb, sem_i, sem_d):
        wid = lax.axis_index("s") * 2 + lax.axis_index("c")
        zv = jnp.zeros((16,), F32)
        for cb in range(8):
            accb[pl.ds(cb * 16, 16)] = zv

        def outer(it, carry):
            chunk = it * NW + wid

            @pl.when(chunk < nchunk)
            def _():
                base = chunk * RC
                hz = pltpu.async_copy(z0h.at[pl.ds(base, RC)], b0, sem_d)
                h0 = pltpu.async_copy(i0h.at[pl.ds(base, RC)], ib0, sem_i)
                h1 = pltpu.async_copy(i1h.at[pl.ds(base, RC)], ib1, sem_i)
                h2 = pltpu.async_copy(i2h.at[pl.ds(base, RC)], ib2, sem_i)
                h0.wait(); h1.wait(); h2.wait()
                g1 = pltpu.async_copy(z1h.at[ib0], b1, sem_d)
                g2 = pltpu.async_copy(z2h.at[ib1], b2, sem_d)
                g3 = pltpu.async_copy(z3h.at[ib2], b3, sem_d)
                hz.wait(); g1.wait(); g2.wait(); g3.wait()

                def row(r, accs):
                    out = list(accs)
                    for cb in range(4):
                        sl = pl.ds(cb * 16, 16)
                        y = b0[r, sl] + b1[r, sl] + b2[r, sl] + b3[r, sl]
                        yb[r, sl] = y
                        out[cb] = accs[cb] + y
                        out[4 + cb] = accs[4 + cb] + y * y
                    return tuple(out)

                accs = lax.fori_loop(0, RC, row, (zv,) * 8)
                pltpu.sync_copy(yb, yh.at[pl.ds(base, RC)])
                for cb in range(8):
                    sl = pl.ds(cb * 16, 16)
                    accb[sl] = accb[sl] + accs[cb]
            return carry

        lax.fori_loop(0, iters, outer, jnp.int32(0))
        pltpu.sync_copy(accb, sth.at[wid])

    return sck


def _aff_pair(p, n):
    se, sf, qe, qf = p[:32], p[32:64], p[64:96], p[96:128]
    me, mf = se / n, sf / n
    ve = qe / n - me * me
    vf = qf / n - mf * mf
    ie = lax.rsqrt(ve + EPS)
    if_ = lax.rsqrt(vf + EPS)
    return jnp.stack([ie, -me * ie, if_, -mf * if_])


def _aff_single(p, n):
    s, q = p[:64], p[64:]
    mean = s / n
    var = q / n - mean * mean
    inv = lax.rsqrt(var + EPS)
    return jnp.stack([inv, -mean * inv])


def kernel(ed, fa, ad, W_e1, W_e2, W_f1, W_f2, W_mc0, W_mc1, W_mc2, W_pb,
           W_fn0, b_fn0, W_fn1, b_fn1, W_fn2, b_fn2):
    B, _, F = ed.shape
    N = B * F
    NT = N // FT

    ed_r = ed.transpose(0, 2, 1).reshape(N, 3)
    fa_r = fa.transpose(0, 2, 1).reshape(N, 3)
    offs = (jnp.arange(B, dtype=jnp.int32) * F)[:, None, None]
    adg = (ad.astype(jnp.int32) + offs).reshape(N, 3)
    i0, i1, i2 = adg[:, 0], adg[:, 1], adg[:, 2]

    row_spec = lambda c: pl.BlockSpec((FT, c), lambda i: (i, 0))
    full2 = lambda a: pl.BlockSpec(a.shape, lambda i: (0,) * a.ndim)
    stat_spec = pl.BlockSpec((1, 1, 128), lambda i: (i, 0, 0))
    stat_shape = jax.ShapeDtypeStruct((NT, 1, 128), F32)

    we1t, wf1t, we2t, wf2t = W_e1.T, W_f1.T, W_e2.T, W_f2.T

    p1 = pl.pallas_call(
        _head_s1_body, grid=(NT,),
        in_specs=[row_spec(3), row_spec(3), full2(we1t), full2(wf1t)],
        out_specs=stat_spec, out_shape=stat_shape,
    )(ed_r, fa_r, we1t, wf1t).sum(axis=(0, 1))
    a1 = _aff_pair(p1, N)

    p2 = pl.pallas_call(
        _head_s2_body, grid=(NT,),
        in_specs=[row_spec(3), row_spec(3), full2(a1),
                  full2(we1t), full2(wf1t), full2(we2t), full2(wf2t)],
        out_specs=stat_spec, out_shape=stat_shape,
    )(ed_r, fa_r, a1, we1t, wf1t, we2t, wf2t).sum(axis=(0, 1))
    a2 = _aff_pair(p2, N)

    z_shape = tuple(jax.ShapeDtypeStruct((N, 64), F32) for _ in range(4))
    z_specs = [row_spec(64)] * 4

    wt0 = jnp.stack([W_mc0[:, :, k].T for k in range(4)])
    zs = pl.pallas_call(
        _head_z_body, grid=(NT,),
        in_specs=[row_spec(3), row_spec(3), full2(a1), full2(a2),
                  full2(we1t), full2(wf1t), full2(we2t), full2(wf2t), full2(wt0)],
        out_specs=z_specs, out_shape=z_shape,
    )(ed_r, fa_r, a1, a2, we1t, wf1t, we2t, wf2t, wt0)

    sc_combine = _make_sc_combine(N)
    for Wn in (W_mc1, W_mc2, None):
        y, st = sc_combine(zs[0], zs[1], zs[2], zs[3], i0, i1, i2)
        aff = _aff_single(st.sum(0), N)
        if Wn is not None:
            wt = jnp.stack([Wn[:, :, k].T for k in range(4)])
            zs = pl.pallas_call(
                _tc_z_body, grid=(NT,),
                in_specs=[row_spec(64), full2(aff), full2(wt)],
                out_specs=z_specs, out_shape=z_shape,
            )(y, aff, wt)

    wpbt = W_pb.T
    pp = pl.pallas_call(
        _pool_body, grid=(NT,),
        in_specs=[row_spec(64), full2(aff), full2(wpbt)],
        out_specs=stat_spec, out_shape=stat_shape,
    )(y, aff, wpbt)
    sb = pp[:, 0, :64].reshape(B, NT // B, 64).sum(1)
    qg = pp[:, 0, 64:].sum(0).reshape(1, 64)
    nf = jnp.array([[1.0 / F, 1.0 / N]], F32)

    b0r, b1r, b2r = b_fn0.reshape(1, -1), b_fn1.reshape(1, -1), b_fn2.reshape(1, -1)
    args = (nf, sb, qg, W_fn0, b0r, W_fn1, b1r, W_fn2, b2r)
    out = pl.pallas_call(
        _fc_body, grid=(1,),
        in_specs=[full2(a) for a in args],
        out_specs=pl.BlockSpec((B, 40), lambda i: (0, 0)),
        out_shape=jax.ShapeDtypeStruct((B, 40), F32),
    )(*args)
    return out


# speedup vs baseline: 9.2506x; 1.0403x over previous
import functools

import jax
import jax.numpy as jnp
from jax import lax
from jax.experimental import pallas as pl
from jax.experimental.pallas import tpu as pltpu
from jax.experimental.pallas import tpu_sc as plsc

F32 = jnp.float32
EPS = 1e-5
FT = 5000


def _dot(a, b):
    return lax.dot_general(a, b, (((1,), (0,)), ((), ())),
                           preferred_element_type=F32)


def _sumsq_row(y):
    p = jnp.concatenate([y, y * y], axis=1)
    return jnp.sum(p, axis=0, keepdims=True)


def _head_s1_body(e_ref, f_ref, we1_ref, wf1_ref, o_ref):
    y = jnp.concatenate([_dot(e_ref[...], we1_ref[...]),
                         _dot(f_ref[...], wf1_ref[...])], axis=1)
    o_ref[0] = _sumsq_row(y)


def _head_s2_body(e_ref, f_ref, a1_ref, we1_ref, wf1_ref, we2_ref, wf2_ref, o_ref):
    x1e = jnp.maximum(_dot(e_ref[...], we1_ref[...]) * a1_ref[0:1] + a1_ref[1:2], 0.0)
    x1f = jnp.maximum(_dot(f_ref[...], wf1_ref[...]) * a1_ref[2:3] + a1_ref[3:4], 0.0)
    y = jnp.concatenate([_dot(x1e, we2_ref[...]), _dot(x1f, wf2_ref[...])], axis=1)
    o_ref[0] = _sumsq_row(y)


def _head_z_body(e_ref, f_ref, a1_ref, a2_ref, we1_ref, wf1_ref,
                 we2_ref, wf2_ref, wt_ref, z0_ref, z1_ref, z2_ref, z3_ref):
    x1e = jnp.maximum(_dot(e_ref[...], we1_ref[...]) * a1_ref[0:1] + a1_ref[1:2], 0.0)
    x1f = jnp.maximum(_dot(f_ref[...], wf1_ref[...]) * a1_ref[2:3] + a1_ref[3:4], 0.0)
    x2e = jnp.maximum(_dot(x1e, we2_ref[...]) * a2_ref[0:1] + a2_ref[1:2], 0.0)
    x2f = jnp.maximum(_dot(x1f, wf2_ref[...]) * a2_ref[2:3] + a2_ref[3:4], 0.0)
    a = jnp.concatenate([x2e, x2f], axis=1)
    z0_ref[...] = _dot(a, wt_ref[0])
    z1_ref[...] = _dot(a, wt_ref[1])
    z2_ref[...] = _dot(a, wt_ref[2])
    z3_ref[...] = _dot(a, wt_ref[3])


def _tc_z_body(y_ref, aff_ref, wt_ref, z0_ref, z1_ref, z2_ref, z3_ref):
    a = jnp.maximum(y_ref[...] * aff_ref[0:1] + aff_ref[1:2], 0.0)
    z0_ref[...] = _dot(a, wt_ref[0])
    z1_ref[...] = _dot(a, wt_ref[1])
    z2_ref[...] = _dot(a, wt_ref[2])
    z3_ref[...] = _dot(a, wt_ref[3])


def _pool_body(y_ref, aff_ref, wpb_ref, o_ref):
    a = jnp.maximum(y_ref[...] * aff_ref[0:1] + aff_ref[1:2], 0.0)
    ypb = _dot(a, wpb_ref[...])
    o_ref[0] = _sumsq_row(ypb)


def _fc_body(nf_ref, sb_ref, q_ref, w0_ref, b0_ref, w1_ref, b1_ref,
             w2_ref, b2_ref, o_ref):
    inv_f = nf_ref[0, 0]
    inv_n = nf_ref[0, 1]
    sb = sb_ref[...]
    t = sb * inv_f
    m = jnp.sum(sb, axis=0, keepdims=True) * inv_n
    var = q_ref[...] * inv_n - m * m
    pooled = (t - m) * lax.rsqrt(var + EPS)
    h = jnp.maximum(lax.dot_general(pooled, w0_ref[...], (((1,), (1,)), ((), ())),
                                    preferred_element_type=F32) + b0_ref[...], 0.0)
    h = jnp.maximum(lax.dot_general(h, w1_ref[...], (((1,), (1,)), ((), ())),
                                    preferred_element_type=F32) + b1_ref[...], 0.0)
    o_ref[...] = lax.dot_general(h, w2_ref[...], (((1,), (1,)), ((), ())),
                                 preferred_element_type=F32) + b2_ref[...]


def _make_sc_combine(N):
    NW = 32
    RC = 128
    nchunk = N // RC
    iters = (nchunk + NW - 1) // NW
    mesh = plsc.VectorSubcoreMesh(core_axis_name="c", subcore_axis_name="s")

    @functools.partial(
        pl.kernel, mesh=mesh,
        compiler_params=pltpu.CompilerParams(use_tc_tiling_on_sc=False),
        out_type=(jax.ShapeDtypeStruct((N, 64), F32),
                  jax.ShapeDtypeStruct((NW, 128), F32)),
        scratch_types=[pltpu.VMEM((RC,), jnp.int32),
                       pltpu.VMEM((RC,), jnp.int32),
                       pltpu.VMEM((RC,), jnp.int32),
                       pltpu.VMEM((RC, 64), F32),
                       pltpu.VMEM((RC, 64), F32),
                       pltpu.VMEM((RC, 64), F32),
                       pltpu.VMEM((RC, 64), F32),
                       pltpu.VMEM((RC, 64), F32),
                       pltpu.VMEM((128,), F32),
                       pltpu.SemaphoreType.DMA,
                       pltpu.SemaphoreType.DMA],
    )
    def sck(z0h, z1h, z2h, z3h, i0h, i1h, i2h, yh, sth,
            ib0, ib1, ib2, b0, b1, b2, b3, yb, accb, sem_i, sem_d):
        wid = lax.axis_index("s") * 2 + lax.axis_index("c")
        zv = jnp.zeros((16,), F32)
        for cb in range(8):
            accb[pl.ds(cb * 16, 16)] = zv

        def outer(it, carry):
            chunk = it * NW + wid

            @pl.when(chunk < nchunk)
            def _():
                base = chunk * RC
                hz = pltpu.async_copy(z0h.at[pl.ds(base, RC)], b0, sem_d)
                h0 = pltpu.async_copy(i0h.at[pl.ds(base, RC)], ib0, sem_i)
                h1 = pltpu.async_copy(i1h.at[pl.ds(base, RC)], ib1, sem_i)
                h2 = pltpu.async_copy(i2h.at[pl.ds(base, RC)], ib2, sem_i)
                h0.wait(); h1.wait(); h2.wait()
                g1 = pltpu.async_copy(z1h.at[ib0], b1, sem_d)
                g2 = pltpu.async_copy(z2h.at[ib1], b2, sem_d)
                g3 = pltpu.async_copy(z3h.at[ib2], b3, sem_d)
                hz.wait(); g1.wait(); g2.wait(); g3.wait()

                def row(r, accs):
                    out = list(accs)
                    for cb in range(4):
                        sl = pl.ds(cb * 16, 16)
                        y = b0[r, sl] + b1[r, sl] + b2[r, sl] + b3[r, sl]
                        yb[r, sl] = y
                        out[cb] = accs[cb] + y
                        out[4 + cb] = accs[4 + cb] + y * y
                    return tuple(out)

                accs = lax.fori_loop(0, RC, row, (zv,) * 8)
                pltpu.sync_copy(yb, yh.at[pl.ds(base, RC)])
                for cb in range(8):
                    sl = pl.ds(cb * 16, 16)
                    accb[sl] = accb[sl] + accs[cb]
            return carry

        lax.fori_loop(0, iters, outer, jnp.int32(0))
        pltpu.sync_copy(accb, sth.at[wid])

    return sck


def _aff_pair(p, n):
    se, sf, qe, qf = p[:32], p[32:64], p[64:96], p[96:128]
    me, mf = se / n, sf / n
    ve = qe / n - me * me
    vf = qf / n - mf * mf
    ie = lax.rsqrt(ve + EPS)
    if_ = lax.rsqrt(vf + EPS)
    return jnp.stack([ie, -me * ie, if_, -mf * if_])


def _aff_single(p, n):
    s, q = p[:64], p[64:]
    mean = s / n
    var = q / n - mean * mean
    inv = lax.rsqrt(var + EPS)
    return jnp.stack([inv, -mean * inv])


def kernel(ed, fa, ad, W_e1, W_e2, W_f1, W_f2, W_mc0, W_mc1, W_mc2, W_pb,
           W_fn0, b_fn0, W_fn1, b_fn1, W_fn2, b_fn2):
    B, _, F = ed.shape
    N = B * F
    NT = N // FT

    ed_r = ed.transpose(0, 2, 1).reshape(N, 3)
    fa_r = fa.transpose(0, 2, 1).reshape(N, 3)
    offs = (jnp.arange(B, dtype=jnp.int32) * F)[:, None, None]
    adg = (ad.astype(jnp.int32) + offs).reshape(N, 3)
    i0, i1, i2 = adg[:, 0], adg[:, 1], adg[:, 2]

    row_spec = lambda c: pl.BlockSpec((FT, c), lambda i: (i, 0))
    full2 = lambda a: pl.BlockSpec(a.shape, lambda i: (0,) * a.ndim)
    stat_spec = pl.BlockSpec((1, 1, 128), lambda i: (i, 0, 0))
    stat_shape = jax.ShapeDtypeStruct((NT, 1, 128), F32)

    we1t, wf1t, we2t, wf2t = W_e1.T, W_f1.T, W_e2.T, W_f2.T

    p1 = pl.pallas_call(
        _head_s1_body, grid=(NT,),
        in_specs=[row_spec(3), row_spec(3), full2(we1t), full2(wf1t)],
        out_specs=stat_spec, out_shape=stat_shape,
    )(ed_r, fa_r, we1t, wf1t).sum(axis=(0, 1))
    a1 = _aff_pair(p1, N)

    p2 = pl.pallas_call(
        _head_s2_body, grid=(NT,),
        in_specs=[row_spec(3), row_spec(3), full2(a1),
                  full2(we1t), full2(wf1t), full2(we2t), full2(wf2t)],
        out_specs=stat_spec, out_shape=stat_shape,
    )(ed_r, fa_r, a1, we1t, wf1t, we2t, wf2t).sum(axis=(0, 1))
    a2 = _aff_pair(p2, N)

    z_shape = tuple(jax.ShapeDtypeStruct((N, 64), F32) for _ in range(4))
    z_specs = [row_spec(64)] * 4

    wt0 = jnp.stack([W_mc0[:, :, k].T for k in range(4)])
    zs = pl.pallas_call(
        _head_z_body, grid=(NT,),
        in_specs=[row_spec(3), row_spec(3), full2(a1), full2(a2),
                  full2(we1t), full2(wf1t), full2(we2t), full2(wf2t), full2(wt0)],
        out_specs=z_specs, out_shape=z_shape,
    )(ed_r, fa_r, a1, a2, we1t, wf1t, we2t, wf2t, wt0)

    sc_combine = _make_sc_combine(N)
    for Wn in (W_mc1, W_mc2, None):
        y, st = sc_combine(zs[0], zs[1], zs[2], zs[3], i0, i1, i2)
        aff = _aff_single(st.sum(0), N)
        if Wn is not None:
            wt = jnp.stack([Wn[:, :, k].T for k in range(4)])
            zs = pl.pallas_call(
                _tc_z_body, grid=(NT,),
                in_specs=[row_spec(64), full2(aff), full2(wt)],
                out_specs=z_specs, out_shape=z_shape,
            )(y, aff, wt)

    wpbt = W_pb.T
    pp = pl.pallas_call(
        _pool_body, grid=(NT,),
        in_specs=[row_spec(64), full2(aff), full2(wpbt)],
        out_specs=stat_spec, out_shape=stat_shape,
    )(y, aff, wpbt)
    sb = pp[:, 0, :64].reshape(B, NT // B, 64).sum(1)
    qg = pp[:, 0, 64:].sum(0).reshape(1, 64)
    nf = jnp.array([[1.0 / F, 1.0 / N]], F32)

    b0r, b1r, b2r = b_fn0.reshape(1, -1), b_fn1.reshape(1, -1), b_fn2.reshape(1, -1)
    args = (nf, sb, qg, W_fn0, b0r, W_fn1, b1r, W_fn2, b2r)
    out = pl.pallas_call(
        _fc_body, grid=(1,),
        in_specs=[full2(a) for a in args],
        out_specs=pl.BlockSpec((B, 40), lambda i: (0, 0)),
        out_shape=jax.ShapeDtypeStruct((B, 40), F32),
    )(*args)
    return out


# speedup vs baseline: 10.1239x; 1.0944x over previous
import functools

import jax
import jax.numpy as jnp
from jax import lax
from jax.experimental import pallas as pl
from jax.experimental.pallas import tpu as pltpu
from jax.experimental.pallas import tpu_sc as plsc

F32 = jnp.float32
EPS = 1e-5
FT = 5000


def _dot(a, b):
    return lax.dot_general(a, b, (((1,), (0,)), ((), ())),
                           preferred_element_type=F32)


def _sumsq_row(y):
    p = jnp.concatenate([y, y * y], axis=1)
    return jnp.sum(p, axis=0, keepdims=True)


def _head_s1_body(e_ref, f_ref, we1_ref, wf1_ref, o_ref):
    y = jnp.concatenate([_dot(e_ref[...], we1_ref[...]),
                         _dot(f_ref[...], wf1_ref[...])], axis=1)
    o_ref[0] = _sumsq_row(y)


def _head_s2_body(e_ref, f_ref, a1_ref, we1_ref, wf1_ref, we2_ref, wf2_ref, o_ref):
    x1e = jnp.maximum(_dot(e_ref[...], we1_ref[...]) * a1_ref[0:1] + a1_ref[1:2], 0.0)
    x1f = jnp.maximum(_dot(f_ref[...], wf1_ref[...]) * a1_ref[2:3] + a1_ref[3:4], 0.0)
    y = jnp.concatenate([_dot(x1e, we2_ref[...]), _dot(x1f, wf2_ref[...])], axis=1)
    o_ref[0] = _sumsq_row(y)


def _head_z_body(e_ref, f_ref, a1_ref, a2_ref, we1_ref, wf1_ref,
                 we2_ref, wf2_ref, wt_ref, z0_ref, z1_ref, z2_ref, z3_ref):
    x1e = jnp.maximum(_dot(e_ref[...], we1_ref[...]) * a1_ref[0:1] + a1_ref[1:2], 0.0)
    x1f = jnp.maximum(_dot(f_ref[...], wf1_ref[...]) * a1_ref[2:3] + a1_ref[3:4], 0.0)
    x2e = jnp.maximum(_dot(x1e, we2_ref[...]) * a2_ref[0:1] + a2_ref[1:2], 0.0)
    x2f = jnp.maximum(_dot(x1f, wf2_ref[...]) * a2_ref[2:3] + a2_ref[3:4], 0.0)
    a = jnp.concatenate([x2e, x2f], axis=1)
    z0_ref[...] = _dot(a, wt_ref[0])
    z1_ref[...] = _dot(a, wt_ref[1])
    z2_ref[...] = _dot(a, wt_ref[2])
    z3_ref[...] = _dot(a, wt_ref[3])


def _tc_z_body(y_ref, aff_ref, wt_ref, z0_ref, z1_ref, z2_ref, z3_ref):
    a = jnp.maximum(y_ref[...] * aff_ref[0:1] + aff_ref[1:2], 0.0)
    z0_ref[...] = _dot(a, wt_ref[0])
    z1_ref[...] = _dot(a, wt_ref[1])
    z2_ref[...] = _dot(a, wt_ref[2])
    z3_ref[...] = _dot(a, wt_ref[3])


def _pool_body(y_ref, aff_ref, wpb_ref, o_ref):
    a = jnp.maximum(y_ref[...] * aff_ref[0:1] + aff_ref[1:2], 0.0)
    ypb = _dot(a, wpb_ref[...])
    o_ref[0] = _sumsq_row(ypb)


def _fc_body(nf_ref, sb_ref, q_ref, w0_ref, b0_ref, w1_ref, b1_ref,
             w2_ref, b2_ref, o_ref):
    inv_f = nf_ref[0, 0]
    inv_n = nf_ref[0, 1]
    sb = sb_ref[...]
    t = sb * inv_f
    m = jnp.sum(sb, axis=0, keepdims=True) * inv_n
    var = q_ref[...] * inv_n - m * m
    pooled = (t - m) * lax.rsqrt(var + EPS)
    h = jnp.maximum(lax.dot_general(pooled, w0_ref[...], (((1,), (1,)), ((), ())),
                                    preferred_element_type=F32) + b0_ref[...], 0.0)
    h = jnp.maximum(lax.dot_general(h, w1_ref[...], (((1,), (1,)), ((), ())),
                                    preferred_element_type=F32) + b1_ref[...], 0.0)
    o_ref[...] = lax.dot_general(h, w2_ref[...], (((1,), (1,)), ((), ())),
                                 preferred_element_type=F32) + b2_ref[...]


def _make_sc_combine(N):
    NW = 32
    RC = 128
    nchunk = N // RC
    iters = (nchunk + NW - 1) // NW
    mesh = plsc.VectorSubcoreMesh(core_axis_name="c", subcore_axis_name="s")

    @functools.partial(
        pl.kernel, mesh=mesh,
        compiler_params=pltpu.CompilerParams(use_tc_tiling_on_sc=False),
        out_type=(jax.ShapeDtypeStruct((N, 64), F32),
                  jax.ShapeDtypeStruct((NW, 128), F32)),
        scratch_types=[pltpu.VMEM((2, RC), jnp.int32),
                       pltpu.VMEM((2, RC), jnp.int32),
                       pltpu.VMEM((2, RC), jnp.int32),
                       pltpu.VMEM((2, RC, 64), F32),
                       pltpu.VMEM((2, RC, 64), F32),
                       pltpu.VMEM((2, RC, 64), F32),
                       pltpu.VMEM((2, RC, 64), F32),
                       pltpu.VMEM((2, RC, 64), F32),
                       pltpu.VMEM((128,), F32),
                       pltpu.SemaphoreType.DMA,
                       pltpu.SemaphoreType.DMA,
                       pltpu.SemaphoreType.DMA],
    )
    def sck(z0h, z1h, z2h, z3h, i0h, i1h, i2h, yh, sth,
            ib0, ib1, ib2, b0, b1, b2, b3, yb, accb, sem_i, sem_d0, sem_d1):
        wid = lax.axis_index("s") * 2 + lax.axis_index("c")
        zv = jnp.zeros((16,), F32)
        for cb in range(8):
            accb[pl.ds(cb * 16, 16)] = zv

        sems = (sem_d0, sem_d1)

        def issue(it, p):
            chunk = it * NW + wid

            @pl.when(chunk < nchunk)
            def _():
                base = chunk * RC
                h0 = pltpu.async_copy(i0h.at[pl.ds(base, RC)], ib0.at[p], sem_i)
                h1 = pltpu.async_copy(i1h.at[pl.ds(base, RC)], ib1.at[p], sem_i)
                h2 = pltpu.async_copy(i2h.at[pl.ds(base, RC)], ib2.at[p], sem_i)
                pltpu.async_copy(z0h.at[pl.ds(base, RC)], b0.at[p], sems[p])
                h0.wait(); h1.wait(); h2.wait()
                pltpu.async_copy(z1h.at[ib0.at[p]], b1.at[p], sems[p])
                pltpu.async_copy(z2h.at[ib1.at[p]], b2.at[p], sems[p])
                pltpu.async_copy(z3h.at[ib2.at[p]], b3.at[p], sems[p])

        def consume(it, p):
            chunk = it * NW + wid

            @pl.when(chunk < nchunk)
            def _():
                base = chunk * RC
                pltpu.make_async_copy(z0h.at[pl.ds(base, RC)], b0.at[p], sems[p]).wait()
                pltpu.make_async_copy(z1h.at[ib0.at[p]], b1.at[p], sems[p]).wait()
                pltpu.make_async_copy(z2h.at[ib1.at[p]], b2.at[p], sems[p]).wait()
                pltpu.make_async_copy(z3h.at[ib2.at[p]], b3.at[p], sems[p]).wait()

                def row(r, accs):
                    out = list(accs)
                    for cb in range(4):
                        sl = pl.ds(cb * 16, 16)
                        y = b0[p, r, sl] + b1[p, r, sl] + b2[p, r, sl] + b3[p, r, sl]
                        yb[p, r, sl] = y
                        out[cb] = accs[cb] + y
                        out[4 + cb] = accs[4 + cb] + y * y
                    return tuple(out)

                accs = lax.fori_loop(0, RC, row, (zv,) * 8)
                pltpu.sync_copy(yb.at[p], yh.at[pl.ds(base, RC)])
                for cb in range(8):
                    sl = pl.ds(cb * 16, 16)
                    accb[sl] = accb[sl] + accs[cb]

        issue(0, 0)

        def outer(j, carry):
            c0 = 2 * j
            issue(c0 + 1, 1)
            consume(c0, 0)
            issue(c0 + 2, 0)
            consume(c0 + 1, 1)
            return carry

        lax.fori_loop(0, (iters + 1) // 2, outer, jnp.int32(0))
        pltpu.sync_copy(accb, sth.at[wid])

    return sck


def _aff_pair(p, n):
    se, sf, qe, qf = p[:32], p[32:64], p[64:96], p[96:128]
    me, mf = se / n, sf / n
    ve = qe / n - me * me
    vf = qf / n - mf * mf
    ie = lax.rsqrt(ve + EPS)
    if_ = lax.rsqrt(vf + EPS)
    return jnp.stack([ie, -me * ie, if_, -mf * if_])


def _aff_single(p, n):
    s, q = p[:64], p[64:]
    mean = s / n
    var = q / n - mean * mean
    inv = lax.rsqrt(var + EPS)
    return jnp.stack([inv, -mean * inv])


def kernel(ed, fa, ad, W_e1, W_e2, W_f1, W_f2, W_mc0, W_mc1, W_mc2, W_pb,
           W_fn0, b_fn0, W_fn1, b_fn1, W_fn2, b_fn2):
    B, _, F = ed.shape
    N = B * F
    NT = N // FT

    ed_r = ed.transpose(0, 2, 1).reshape(N, 3)
    fa_r = fa.transpose(0, 2, 1).reshape(N, 3)
    offs = (jnp.arange(B, dtype=jnp.int32) * F)[:, None, None]
    adg = (ad.astype(jnp.int32) + offs).reshape(N, 3)
    i0, i1, i2 = adg[:, 0], adg[:, 1], adg[:, 2]

    row_spec = lambda c: pl.BlockSpec((FT, c), lambda i: (i, 0))
    full2 = lambda a: pl.BlockSpec(a.shape, lambda i: (0,) * a.ndim)
    stat_spec = pl.BlockSpec((1, 1, 128), lambda i: (i, 0, 0))
    stat_shape = jax.ShapeDtypeStruct((NT, 1, 128), F32)

    we1t, wf1t, we2t, wf2t = W_e1.T, W_f1.T, W_e2.T, W_f2.T

    p1 = pl.pallas_call(
        _head_s1_body, grid=(NT,),
        in_specs=[row_spec(3), row_spec(3), full2(we1t), full2(wf1t)],
        out_specs=stat_spec, out_shape=stat_shape,
    )(ed_r, fa_r, we1t, wf1t).sum(axis=(0, 1))
    a1 = _aff_pair(p1, N)

    p2 = pl.pallas_call(
        _head_s2_body, grid=(NT,),
        in_specs=[row_spec(3), row_spec(3), full2(a1),
                  full2(we1t), full2(wf1t), full2(we2t), full2(wf2t)],
        out_specs=stat_spec, out_shape=stat_shape,
    )(ed_r, fa_r, a1, we1t, wf1t, we2t, wf2t).sum(axis=(0, 1))
    a2 = _aff_pair(p2, N)

    z_shape = tuple(jax.ShapeDtypeStruct((N, 64), F32) for _ in range(4))
    z_specs = [row_spec(64)] * 4

    wt0 = jnp.stack([W_mc0[:, :, k].T for k in range(4)])
    zs = pl.pallas_call(
        _head_z_body, grid=(NT,),
        in_specs=[row_spec(3), row_spec(3), full2(a1), full2(a2),
                  full2(we1t), full2(wf1t), full2(we2t), full2(wf2t), full2(wt0)],
        out_specs=z_specs, out_shape=z_shape,
    )(ed_r, fa_r, a1, a2, we1t, wf1t, we2t, wf2t, wt0)

    sc_combine = _make_sc_combine(N)
    for Wn in (W_mc1, W_mc2, None):
        y, st = sc_combine(zs[0], zs[1], zs[2], zs[3], i0, i1, i2)
        aff = _aff_single(st.sum(0), N)
        if Wn is not None:
            wt = jnp.stack([Wn[:, :, k].T for k in range(4)])
            zs = pl.pallas_call(
                _tc_z_body, grid=(NT,),
                in_specs=[row_spec(64), full2(aff), full2(wt)],
                out_specs=z_specs, out_shape=z_shape,
            )(y, aff, wt)

    wpbt = W_pb.T
    pp = pl.pallas_call(
        _pool_body, grid=(NT,),
        in_specs=[row_spec(64), full2(aff), full2(wpbt)],
        out_specs=stat_spec, out_shape=stat_shape,
    )(y, aff, wpbt)
    sb = pp[:, 0, :64].reshape(B, NT // B, 64).sum(1)
    qg = pp[:, 0, 64:].sum(0).reshape(1, 64)
    nf = jnp.array([[1.0 / F, 1.0 / N]], F32)

    b0r, b1r, b2r = b_fn0.reshape(1, -1), b_fn1.reshape(1, -1), b_fn2.reshape(1, -1)
    args = (nf, sb, qg, W_fn0, b0r, W_fn1, b1r, W_fn2, b2r)
    out = pl.pallas_call(
        _fc_body, grid=(1,),
        in_specs=[full2(a) for a in args],
        out_specs=pl.BlockSpec((B, 40), lambda i: (0, 0)),
        out_shape=jax.ShapeDtypeStruct((B, 40), F32),
    )(*args)
    return out
